# baseline (speedup 1.0000x reference)
import functools

import jax
import jax.numpy as jnp
from jax.experimental import pallas as pl

_B = 4
_C = 384
_HW = 224 * 224
_R = 48
_E = 64
_K = 8
_EPS = 1e-5

_ROWS_PER_BLOCK = 64


def _pool_body(x_ref, o_ref):
    o_ref[...] = jnp.sum(x_ref[...], axis=1, keepdims=True)


def _route_body(ps_ref, w1_ref, b1_ref, w2_ref, b2_ref, vals_ref, idx_ref):
    pooled = ps_ref[...]
    h = jax.lax.dot_general(pooled, w1_ref[...], (((1,), (1,)), ((), ())),
                            preferred_element_type=jnp.float32)
    h = h + b1_ref[...]
    h = h * jax.nn.sigmoid(h)
    logits = jax.lax.dot_general(h, w2_ref[...], (((1,), (1,)), ((), ())),
                                 preferred_element_type=jnp.float32)
    logits = logits + b2_ref[...]
    m = jnp.max(logits, axis=1, keepdims=True)
    e = jnp.exp(logits - m)
    probs = e / jnp.sum(e, axis=1, keepdims=True)

    iota = jax.lax.broadcasted_iota(jnp.int32, (_B, _E), 1)
    p = probs
    vals = []
    idxs = []
    for _ in range(_K):
        mx = jnp.max(p, axis=1, keepdims=True)
        sel = jnp.min(jnp.where(p == mx, iota, _E), axis=1, keepdims=True)
        vals.append(mx)
        idxs.append(sel)
        p = jnp.where(iota == sel, -jnp.inf, p)
    v = jnp.concatenate(vals, axis=1)
    i = jnp.concatenate(idxs, axis=1)
    s = jnp.sum(v, axis=1, keepdims=True) + 1e-6
    vals_ref[...] = v / s
    idx_ref[...] = i


@jax.jit
def kernel(x, W1, gamma1, beta1, mean1, var1, W2, gamma2, beta2, mean2, var2):
    s1 = gamma1 * jax.lax.rsqrt(var1 + _EPS)
    s2 = gamma2 * jax.lax.rsqrt(var2 + _EPS)
    w1 = (W1 * s1[:, None]) * (1.0 / _HW)
    b1 = (beta1 - mean1 * s1)[None, :]
    w2 = W2 * s2[:, None]
    b2 = (beta2 - mean2 * s2)[None, :]

    rows = _B * _C
    xv = x.reshape(rows, _HW)
    n_blocks = rows // _ROWS_PER_BLOCK

    sums = pl.pallas_call(
        _pool_body,
        grid=(n_blocks,),
        in_specs=[pl.BlockSpec((_ROWS_PER_BLOCK, _HW), lambda i: (i, 0))],
        out_specs=pl.BlockSpec((_ROWS_PER_BLOCK, 1), lambda i: (i, 0)),
        out_shape=jax.ShapeDtypeStruct((rows, 1), jnp.float32),
    )(xv)

    pooled_sums = sums.reshape(_B, _C)

    vals, idxs = pl.pallas_call(
        _route_body,
        out_shape=(
            jax.ShapeDtypeStruct((_B, _K), jnp.float32),
            jax.ShapeDtypeStruct((_B, _K), jnp.int32),
        ),
    )(pooled_sums, w1, b1, w2, b2)
    return vals, idxs


# speedup vs baseline: 1.6183x; 1.6183x over previous
import functools

import jax
import jax.numpy as jnp
from jax.experimental import pallas as pl

_B = 4
_C = 384
_HW = 224 * 224
_R = 48
_E = 64
_K = 8
_EPS = 1e-5

_CBLK = 32


def _pool_body(x_ref, o_ref):
    s = jnp.sum(x_ref[0], axis=1)
    o_ref[0, 0, :] = jnp.sum(s, axis=1)


def _route_body(ps_ref, w1_ref, b1_ref, w2_ref, b2_ref, vals_ref, idx_ref):
    pooled = ps_ref[...]
    h = jax.lax.dot_general(pooled, w1_ref[...], (((1,), (1,)), ((), ())),
                            preferred_element_type=jnp.float32)
    h = h + b1_ref[...]
    h = h * jax.nn.sigmoid(h)
    logits = jax.lax.dot_general(h, w2_ref[...], (((1,), (1,)), ((), ())),
                                 preferred_element_type=jnp.float32)
    logits = logits + b2_ref[...]
    m = jnp.max(logits, axis=1, keepdims=True)
    e = jnp.exp(logits - m)
    probs = e / jnp.sum(e, axis=1, keepdims=True)

    iota = jax.lax.broadcasted_iota(jnp.int32, (_B, _E), 1)
    p = probs
    vals = []
    idxs = []
    for _ in range(_K):
        mx = jnp.max(p, axis=1, keepdims=True)
        sel = jnp.min(jnp.where(p == mx, iota, _E), axis=1, keepdims=True)
        vals.append(mx)
        idxs.append(sel)
        p = jnp.where(iota == sel, -jnp.inf, p)
    v = jnp.concatenate(vals, axis=1)
    i = jnp.concatenate(idxs, axis=1)
    s = jnp.sum(v, axis=1, keepdims=True) + 1e-6
    vals_ref[...] = v / s
    idx_ref[...] = i


@jax.jit
def kernel(x, W1, gamma1, beta1, mean1, var1, W2, gamma2, beta2, mean2, var2):
    s1 = gamma1 * jax.lax.rsqrt(var1 + _EPS)
    s2 = gamma2 * jax.lax.rsqrt(var2 + _EPS)
    w1 = (W1 * s1[:, None]) * (1.0 / _HW)
    b1 = (beta1 - mean1 * s1)[None, :]
    w2 = W2 * s2[:, None]
    b2 = (beta2 - mean2 * s2)[None, :]

    n_cblk = _C // _CBLK
    pooled_sums = pl.pallas_call(
        _pool_body,
        grid=(_B, n_cblk),
        in_specs=[pl.BlockSpec((1, _CBLK, 224, 224), lambda b, c: (b, c, 0, 0))],
        out_specs=pl.BlockSpec((1, 1, _CBLK), lambda b, c: (b * n_cblk + c, 0, 0)),
        out_shape=jax.ShapeDtypeStruct((_B * n_cblk, 1, _CBLK), jnp.float32),
    )(x).reshape(_B, _C)

    vals, idxs = pl.pallas_call(
        _route_body,
        out_shape=(
            jax.ShapeDtypeStruct((_B, _K), jnp.float32),
            jax.ShapeDtypeStruct((_B, _K), jnp.int32),
        ),
    )(pooled_sums, w1, b1, w2, b2)
    return vals, idxs
